# baseline (speedup 1.0000x reference)
import numpy as np
import jax
import jax.numpy as jnp
from jax.experimental import pallas as pl
from jax.experimental.pallas import tpu as pltpu

_C, _T, _H, _W = 3, 32, 384, 384
_ALPHA = 4
_NSLOW = _T // _ALPHA
_IDX = tuple(int(v) for v in np.linspace(0.0, _T - 1, _NSLOW).astype(np.float32))

_FCHUNK = 8


def _dma_body(in_ref, slow_ref, fast_ref, fast_sem, slow_sem):
    fast_copies = []
    for c in range(_C):
        for g in range(_T // _FCHUNK):
            fast_copies.append(pltpu.make_async_copy(
                in_ref.at[c, pl.ds(g * _FCHUNK, _FCHUNK)],
                fast_ref.at[c, pl.ds(g * _FCHUNK, _FCHUNK)],
                fast_sem,
            ))
    slow_copies = []
    for c in range(_C):
        for s in range(_NSLOW):
            slow_copies.append(pltpu.make_async_copy(
                in_ref.at[c, pl.ds(_IDX[s], 1)],
                slow_ref.at[c, pl.ds(s, 1)],
                slow_sem,
            ))
    for cp in fast_copies:
        cp.start()
    for cp in slow_copies:
        cp.start()
    for cp in fast_copies:
        cp.wait()
    for cp in slow_copies:
        cp.wait()


def kernel(frames):
    slow, fast = pl.pallas_call(
        _dma_body,
        in_specs=[pl.BlockSpec(memory_space=pl.ANY)],
        out_specs=[
            pl.BlockSpec(memory_space=pl.ANY),
            pl.BlockSpec(memory_space=pl.ANY),
        ],
        out_shape=[
            jax.ShapeDtypeStruct((_C, _NSLOW, _H, _W), frames.dtype),
            jax.ShapeDtypeStruct((_C, _T, _H, _W), frames.dtype),
        ],
        scratch_shapes=[pltpu.SemaphoreType.DMA, pltpu.SemaphoreType.DMA],
    )(frames)
    return (slow, fast)


# speedup vs baseline: 27.0734x; 27.0734x over previous
import functools
import operator

import numpy as np
import jax
import jax.numpy as jnp
from jax.experimental import pallas as pl

_C, _T, _H, _W = 3, 32, 384, 384
_ALPHA = 4
_NSLOW = _T // _ALPHA
_IDX = tuple(int(v) for v in np.linspace(0.0, _T - 1, _NSLOW).astype(np.float32))


def _body(in_ref, slow_ref, fast_ref):
    t = pl.program_id(1)
    x = in_ref[...]
    fast_ref[...] = x
    sel = functools.reduce(operator.or_, [t == i for i in _IDX])

    @pl.when(sel)
    def _():
        slow_ref[...] = x


def _slow_index_map(c, t):
    slot = sum((t >= i).astype(jnp.int32) for i in _IDX[1:])
    return (c, slot, 0, 0)


def kernel(frames):
    slow, fast = pl.pallas_call(
        _body,
        grid=(_C, _T),
        in_specs=[pl.BlockSpec((1, 1, _H, _W), lambda c, t: (c, t, 0, 0))],
        out_specs=[
            pl.BlockSpec((1, 1, _H, _W), _slow_index_map),
            pl.BlockSpec((1, 1, _H, _W), lambda c, t: (c, t, 0, 0)),
        ],
        out_shape=[
            jax.ShapeDtypeStruct((_C, _NSLOW, _H, _W), frames.dtype),
            jax.ShapeDtypeStruct((_C, _T, _H, _W), frames.dtype),
        ],
    )(frames)
    return (slow, fast)


# speedup vs baseline: 52.7413x; 1.9481x over previous
import numpy as np
import jax
import jax.numpy as jnp
from jax.experimental import pallas as pl

_C, _T, _H, _W = 3, 32, 384, 384
_ALPHA = 4
_NSLOW = _T // _ALPHA
_IDX = tuple(int(v) for v in np.linspace(0.0, _T - 1, _NSLOW).astype(np.float32))
_QUAD = 4
assert all(_IDX[q] // _QUAD == q for q in range(_NSLOW))


def _body(in_ref, slow_ref, fast_ref):
    q = pl.program_id(0)
    x = in_ref[...]
    fast_ref[...] = x
    off = _IDX[-1] * q // (_NSLOW - 1) - _QUAD * q
    slow_ref[...] = in_ref[:, pl.ds(off, 1)]


def kernel(frames):
    slow, fast = pl.pallas_call(
        _body,
        grid=(_NSLOW,),
        in_specs=[pl.BlockSpec((_C, _QUAD, _H, _W), lambda q: (0, q, 0, 0))],
        out_specs=[
            pl.BlockSpec((_C, 1, _H, _W), lambda q: (0, q, 0, 0)),
            pl.BlockSpec((_C, _QUAD, _H, _W), lambda q: (0, q, 0, 0)),
        ],
        out_shape=[
            jax.ShapeDtypeStruct((_C, _NSLOW, _H, _W), frames.dtype),
            jax.ShapeDtypeStruct((_C, _T, _H, _W), frames.dtype),
        ],
    )(frames)
    return (slow, fast)


# speedup vs baseline: 54.4928x; 1.0332x over previous
import numpy as np
import jax
import jax.numpy as jnp
from jax.experimental import pallas as pl
from jax.experimental.pallas import tpu as pltpu

_C, _T, _H, _W = 3, 32, 384, 384
_ALPHA = 4
_NSLOW = _T // _ALPHA
_IDX = tuple(int(v) for v in np.linspace(0.0, _T - 1, _NSLOW).astype(np.float32))
_OCT = 8
assert all(_IDX[2 * o] // _OCT == o and _IDX[2 * o + 1] // _OCT == o
           for o in range(_T // _OCT))
assert all(_IDX[2 * o] - _OCT * o == max(0, o - 1) for o in range(_T // _OCT))
assert all(_IDX[2 * o + 1] - _OCT * o == o + 4 for o in range(_T // _OCT))


def _body(in_ref, slow_ref, fast_ref):
    o = pl.program_id(0)
    x = in_ref[...]
    fast_ref[...] = x
    off0 = jnp.maximum(0, o - 1)
    off1 = o + 4
    slow_ref[:, pl.ds(0, 1)] = in_ref[:, pl.ds(off0, 1)]
    slow_ref[:, pl.ds(1, 1)] = in_ref[:, pl.ds(off1, 1)]


def kernel(frames):
    slow, fast = pl.pallas_call(
        _body,
        grid=(_T // _OCT,),
        in_specs=[pl.BlockSpec((_C, _OCT, _H, _W), lambda o: (0, o, 0, 0))],
        out_specs=[
            pl.BlockSpec((_C, 2, _H, _W), lambda o: (0, o, 0, 0)),
            pl.BlockSpec((_C, _OCT, _H, _W), lambda o: (0, o, 0, 0)),
        ],
        out_shape=[
            jax.ShapeDtypeStruct((_C, _NSLOW, _H, _W), frames.dtype),
            jax.ShapeDtypeStruct((_C, _T, _H, _W), frames.dtype),
        ],
        compiler_params=pltpu.CompilerParams(
            vmem_limit_bytes=100 * 1024 * 1024,
        ),
    )(frames)
    return (slow, fast)
